# baseline (speedup 1.0000x reference)
import jax
import jax.numpy as jnp
from jax.experimental import pallas as pl

_L = 2048
_D = 768
_K = 256
_S = 64
_NT = _L // _S


def _count(pred):
    return jnp.sum(pred.astype(jnp.int32))


def _mask_kernel(x_ref, wrt_ref, br_ref, m_ref):
    xb = x_ref[0]
    w = jnp.dot(xb, wrt_ref[...], preferred_element_type=jnp.float32,
                precision=jax.lax.Precision.HIGHEST)
    w = w + br_ref[0, 0]

    s = jax.lax.bitcast_convert_type(w, jnp.int32)
    skey = jnp.where(s < 0, s ^ jnp.int32(0x7FFFFFFF), s)

    thr = jnp.where(_count(skey >= 0) >= _K, jnp.int32(0), jnp.int32(-(2 ** 31)))
    for bit in range(30, -1, -1):
        cand = thr + jnp.int32(1 << bit)
        thr = jnp.where(_count(skey >= cand) >= _K, cand, thr)

    need = _K - _count(skey > thr)
    tie = skey == thr
    idx = jax.lax.broadcasted_iota(jnp.int32, (_L, 1), 0)
    cut = jnp.int32(0)
    for bit in range(11, -1, -1):
        cand = cut + jnp.int32(1 << bit)
        cut = jnp.where(_count(tie & (idx < cand)) <= need, cand, cut)

    mask = (skey > thr) | (tie & (idx < cut))
    m_ref[0] = mask.astype(jnp.float32)


def _copy_kernel(x_ref, m_ref, out_ref):
    xb = x_ref[0]
    mb = m_ref[0, 0]
    ir = jax.lax.broadcasted_iota(jnp.int32, (_S, _S), 0)
    ic = jax.lax.broadcasted_iota(jnp.int32, (_S, _S), 1)
    p = (ir + ic == _S - 1).astype(jnp.float32)
    flipped = jnp.dot(p, xb, preferred_element_type=jnp.float32,
                      precision=jax.lax.Precision.HIGHEST)
    out_ref[0] = mb * flipped


def kernel(x, Wr, br, Wq, Wk, Wv):
    b, l, d = x.shape
    wrt = Wr.T
    brm = br.reshape(1, 1)
    mask = pl.pallas_call(
        _mask_kernel,
        grid=(b,),
        in_specs=[
            pl.BlockSpec((1, l, d), lambda i: (i, 0, 0)),
            pl.BlockSpec((d, 1), lambda i: (0, 0)),
            pl.BlockSpec((1, 1), lambda i: (0, 0)),
        ],
        out_specs=pl.BlockSpec((1, l, 1), lambda i: (i, 0, 0)),
        out_shape=jax.ShapeDtypeStruct((b, l, 1), jnp.float32),
    )(x, wrt, brm)
    mask4 = mask.reshape(b, _NT, _S, 1)
    return pl.pallas_call(
        _copy_kernel,
        grid=(b, _NT),
        in_specs=[
            pl.BlockSpec((1, _S, d), lambda i, t: (i, _NT - 1 - t, 0)),
            pl.BlockSpec((1, 1, _S, 1), lambda i, t: (i, t, 0, 0)),
        ],
        out_specs=pl.BlockSpec((1, _S, d), lambda i, t: (i, t, 0)),
        out_shape=jax.ShapeDtypeStruct((b, l, d), x.dtype),
    )(x, mask4)


# speedup vs baseline: 1.1073x; 1.1073x over previous
import jax
import jax.numpy as jnp
from jax.experimental import pallas as pl

_L = 2048
_D = 768
_K = 256
_S = 64
_NT = _L // _S


def _count(pred):
    return jnp.sum(pred.astype(jnp.int32))


def _mask_kernel(x_ref, wrt_ref, br_ref, m_ref):
    xb = x_ref[0]
    w = jnp.dot(xb, wrt_ref[...], preferred_element_type=jnp.float32)
    w = w + br_ref[0, 0]

    s = jax.lax.bitcast_convert_type(w, jnp.int32)
    skey = jnp.where(s < 0, s ^ jnp.int32(0x7FFFFFFF), s)

    thr = jnp.where(_count(skey >= 0) >= _K, jnp.int32(0), jnp.int32(-(2 ** 31)))
    for bit in range(30, -1, -1):
        cand = thr + jnp.int32(1 << bit)
        thr = jnp.where(_count(skey >= cand) >= _K, cand, thr)

    need = _K - _count(skey > thr)
    tie = skey == thr
    idx = jax.lax.broadcasted_iota(jnp.int32, (_L, 1), 0)
    cut = jnp.int32(0)
    for bit in range(11, -1, -1):
        cand = cut + jnp.int32(1 << bit)
        cut = jnp.where(_count(tie & (idx < cand)) <= need, cand, cut)

    mask = (skey > thr) | (tie & (idx < cut))
    m_ref[0] = mask.astype(jnp.float32)


def _copy_kernel(x_ref, m_ref, out_ref):
    xb = x_ref[0]
    mb = m_ref[0, 0]
    ir = jax.lax.broadcasted_iota(jnp.int32, (_S, _S), 0)
    ic = jax.lax.broadcasted_iota(jnp.int32, (_S, _S), 1)
    p = (ir + ic == _S - 1).astype(jnp.float32)
    flipped = jnp.dot(p, xb, preferred_element_type=jnp.float32,
                      precision=jax.lax.Precision.HIGHEST)
    out_ref[0] = mb * flipped


def kernel(x, Wr, br, Wq, Wk, Wv):
    b, l, d = x.shape
    wrt = Wr.T
    brm = br.reshape(1, 1)
    mask = pl.pallas_call(
        _mask_kernel,
        grid=(b,),
        in_specs=[
            pl.BlockSpec((1, l, d), lambda i: (i, 0, 0)),
            pl.BlockSpec((d, 1), lambda i: (0, 0)),
            pl.BlockSpec((1, 1), lambda i: (0, 0)),
        ],
        out_specs=pl.BlockSpec((1, l, 1), lambda i: (i, 0, 0)),
        out_shape=jax.ShapeDtypeStruct((b, l, 1), jnp.float32),
    )(x, wrt, brm)
    mask4 = mask.reshape(b, _NT, _S, 1)
    return pl.pallas_call(
        _copy_kernel,
        grid=(b, _NT),
        in_specs=[
            pl.BlockSpec((1, _S, d), lambda i, t: (i, _NT - 1 - t, 0)),
            pl.BlockSpec((1, 1, _S, 1), lambda i, t: (i, t, 0, 0)),
        ],
        out_specs=pl.BlockSpec((1, _S, d), lambda i, t: (i, t, 0)),
        out_shape=jax.ShapeDtypeStruct((b, l, d), x.dtype),
    )(x, mask4)


# speedup vs baseline: 1.5784x; 1.4255x over previous
import jax
import jax.numpy as jnp
from jax.experimental import pallas as pl

_L = 2048
_D = 768
_K = 256
_S = 64
_NT = _L // _S


def _route_kernel(x_ref, wrt_ref, br_ref, out_ref):
    xb = x_ref[0]
    w = jnp.dot(xb, wrt_ref[...], preferred_element_type=jnp.float32)
    w = w + br_ref[0, 0]

    s = jax.lax.bitcast_convert_type(w, jnp.int32)
    skey = jnp.where(s < 0, s ^ jnp.int32(0x7FFFFFFF), s)
    skey_r = skey.reshape(16, 128)

    def count(pred):
        return jnp.sum(pred.astype(jnp.int32))

    thr = jnp.where(count(skey_r >= 0) >= _K, jnp.int32(0), jnp.int32(-(2 ** 31)))
    for bit in range(30, -1, -1):
        cand = thr + jnp.int32(1 << bit)
        thr = jnp.where(count(skey_r >= cand) >= _K, cand, thr)

    need = _K - count(skey_r > thr)
    tie_r = skey_r == thr
    idx_r = (jax.lax.broadcasted_iota(jnp.int32, (16, 128), 0) * 128
             + jax.lax.broadcasted_iota(jnp.int32, (16, 128), 1))
    cut = jnp.int32(0)
    for bit in range(11, -1, -1):
        cand = cut + jnp.int32(1 << bit)
        cut = jnp.where(count(tie_r & (idx_r < cand)) <= need, cand, cut)

    idx = jax.lax.broadcasted_iota(jnp.int32, (_L, 1), 0)
    mask = (skey > thr) | ((skey == thr) & (idx < cut))

    ir = jax.lax.broadcasted_iota(jnp.int32, (_S, _S), 0)
    ic = jax.lax.broadcasted_iota(jnp.int32, (_S, _S), 1)
    p = (ir + ic == _S - 1).astype(jnp.float32)
    zero = jnp.float32(0.0)
    for t in range(_NT):
        src = xb[_L - _S * (t + 1):_L - _S * t, :]
        flipped = jnp.dot(p, src, preferred_element_type=jnp.float32,
                          precision=jax.lax.Precision.HIGHEST)
        mb = mask[_S * t:_S * (t + 1), :]
        out_ref[0, _S * t:_S * (t + 1), :] = jnp.where(mb, flipped, zero)


def kernel(x, Wr, br, Wq, Wk, Wv):
    b, l, d = x.shape
    wrt = Wr.T
    brm = br.reshape(1, 1)
    return pl.pallas_call(
        _route_kernel,
        grid=(b,),
        in_specs=[
            pl.BlockSpec((1, l, d), lambda i: (i, 0, 0)),
            pl.BlockSpec((d, 1), lambda i: (0, 0)),
            pl.BlockSpec((1, 1), lambda i: (0, 0)),
        ],
        out_specs=pl.BlockSpec((1, l, d), lambda i: (i, 0, 0)),
        out_shape=jax.ShapeDtypeStruct((b, l, d), x.dtype),
    )(x, wrt, brm)


# speedup vs baseline: 1.7285x; 1.0951x over previous
import jax
import jax.numpy as jnp
from jax.experimental import pallas as pl
from jax.experimental.pallas import tpu as pltpu

_L = 2048
_D = 768
_K = 256
_S = 64
_NT = _L // _S


def _route_kernel(x_ref, wrt_ref, br_ref, out_ref, skey_scr):
    xb = x_ref[0]
    w = jnp.dot(xb, wrt_ref[...], preferred_element_type=jnp.float32)
    w = w + br_ref[0, 0]

    s = jax.lax.bitcast_convert_type(w, jnp.int32)
    skey = jnp.where(s < 0, s ^ jnp.int32(0x7FFFFFFF), s)
    skey_scr[...] = skey.reshape(16, 128)
    skey_r = skey_scr[...]

    def cntb(pred):
        ps = jnp.sum(pred.astype(jnp.float32), axis=0, keepdims=True)
        for sh in (1, 2, 4, 8, 16, 32, 64):
            ps = ps + pltpu.roll(ps, sh, 1)
        return ps

    kfv = jnp.full((1, 128), float(_K), jnp.float32)
    zero_i = jnp.zeros((1, 128), jnp.int32)
    int_min = jnp.full((1, 128), -(2 ** 31), jnp.int32)

    thr = jnp.where(cntb(skey_r >= 0) >= kfv, zero_i, int_min)
    for bit in range(30, -1, -1):
        cand = thr + jnp.int32(1 << bit)
        thr = jnp.where(cntb(skey_r >= cand) >= kfv, cand, thr)

    need = kfv - cntb(skey_r > thr)
    tie_r = skey_r == thr
    idx_r = (jax.lax.broadcasted_iota(jnp.int32, (16, 128), 0) * 128
             + jax.lax.broadcasted_iota(jnp.int32, (16, 128), 1))
    cut = zero_i
    for bit in range(11, -1, -1):
        cand = cut + jnp.int32(1 << bit)
        cut = jnp.where(cntb(tie_r & (idx_r < cand)) <= need, cand, cut)

    thr_s = thr[0:1, 0:1]
    cut_s = cut[0:1, 0:1]
    idx = jax.lax.broadcasted_iota(jnp.int32, (_L, 1), 0)
    mask = (skey > thr_s) | ((skey == thr_s) & (idx < cut_s))

    ir = jax.lax.broadcasted_iota(jnp.int32, (_S, _S), 0)
    ic = jax.lax.broadcasted_iota(jnp.int32, (_S, _S), 1)
    p = (ir + ic == _S - 1).astype(jnp.float32)
    zero = jnp.float32(0.0)
    for t in range(_NT):
        src = xb[_L - _S * (t + 1):_L - _S * t, :]
        flipped = jnp.dot(p, src, preferred_element_type=jnp.float32,
                          precision=jax.lax.Precision.HIGHEST)
        mb = mask[_S * t:_S * (t + 1), :]
        out_ref[0, _S * t:_S * (t + 1), :] = jnp.where(mb, flipped, zero)


def kernel(x, Wr, br, Wq, Wk, Wv):
    b, l, d = x.shape
    wrt = Wr.T
    brm = br.reshape(1, 1)
    return pl.pallas_call(
        _route_kernel,
        grid=(b,),
        in_specs=[
            pl.BlockSpec((1, l, d), lambda i: (i, 0, 0)),
            pl.BlockSpec((d, 1), lambda i: (0, 0)),
            pl.BlockSpec((1, 1), lambda i: (0, 0)),
        ],
        out_specs=pl.BlockSpec((1, l, d), lambda i: (i, 0, 0)),
        out_shape=jax.ShapeDtypeStruct((b, l, d), x.dtype),
        scratch_shapes=[pltpu.VMEM((16, 128), jnp.int32)],
    )(x, wrt, brm)


# speedup vs baseline: 1.8445x; 1.0671x over previous
import jax
import jax.numpy as jnp
from jax.experimental import pallas as pl
from jax.experimental.pallas import tpu as pltpu

_L = 2048
_D = 768
_K = 256
_S = 64
_NT = _L // _S


def _route_kernel(x_ref, wrt_ref, br_ref, out_ref, skey_scr):
    xb = x_ref[0]
    w = jnp.dot(xb, wrt_ref[...], preferred_element_type=jnp.float32)
    w = w + br_ref[0, 0]

    s = jax.lax.bitcast_convert_type(w, jnp.int32)
    skey = jnp.where(s < 0, s ^ jnp.int32(0x7FFFFFFF), s)
    skey_scr[...] = skey.reshape(16, 128)
    skey_r = skey_scr[...]

    ones16 = jnp.ones((1, 16), jnp.float32)
    ones128 = jnp.ones((128, 128), jnp.float32)

    def cntb(pred):
        p = pred.astype(jnp.float32)
        cs = jnp.dot(ones16, p, preferred_element_type=jnp.float32)
        return jnp.dot(cs, ones128, preferred_element_type=jnp.float32)

    kfv = jnp.full((1, 128), float(_K), jnp.float32)
    zero_i = jnp.zeros((1, 128), jnp.int32)
    int_min = jnp.full((1, 128), -(2 ** 31), jnp.int32)

    thr = jnp.where(cntb(skey_r >= 0) >= kfv, zero_i, int_min)
    for shift, nc in [(28, 8), (24, 16), (20, 16), (16, 16), (12, 16),
                      (8, 16), (4, 16), (0, 16)]:
        d = zero_i
        for c in range(1, nc):
            ok = cntb(skey_r >= thr + jnp.int32(c << shift)) >= kfv
            d = d + ok.astype(jnp.int32)
        thr = thr + (d << shift)

    need = kfv - cntb(skey_r > thr)
    tie_r = skey_r == thr
    idx_r = (jax.lax.broadcasted_iota(jnp.int32, (16, 128), 0) * 128
             + jax.lax.broadcasted_iota(jnp.int32, (16, 128), 1))
    cut = zero_i
    for shift in (8, 4, 0):
        d = zero_i
        for c in range(1, 16):
            ok = cntb(tie_r & (idx_r < cut + jnp.int32(c << shift))) <= need
            d = d + ok.astype(jnp.int32)
        cut = cut + (d << shift)

    thr_s = thr[0:1, 0:1]
    cut_s = cut[0:1, 0:1]
    idx = jax.lax.broadcasted_iota(jnp.int32, (_L, 1), 0)
    mask = (skey > thr_s) | ((skey == thr_s) & (idx < cut_s))

    ir = jax.lax.broadcasted_iota(jnp.int32, (_S, _S), 0)
    ic = jax.lax.broadcasted_iota(jnp.int32, (_S, _S), 1)
    p = (ir + ic == _S - 1).astype(jnp.float32)
    zero = jnp.float32(0.0)
    for t in range(_NT):
        src = xb[_L - _S * (t + 1):_L - _S * t, :]
        flipped = jnp.dot(p, src, preferred_element_type=jnp.float32,
                          precision=jax.lax.Precision.HIGHEST)
        mb = mask[_S * t:_S * (t + 1), :]
        out_ref[0, _S * t:_S * (t + 1), :] = jnp.where(mb, flipped, zero)


def kernel(x, Wr, br, Wq, Wk, Wv):
    b, l, d = x.shape
    wrt = Wr.T
    brm = br.reshape(1, 1)
    return pl.pallas_call(
        _route_kernel,
        grid=(b,),
        in_specs=[
            pl.BlockSpec((1, l, d), lambda i: (i, 0, 0)),
            pl.BlockSpec((d, 1), lambda i: (0, 0)),
            pl.BlockSpec((1, 1), lambda i: (0, 0)),
        ],
        out_specs=pl.BlockSpec((1, l, d), lambda i: (i, 0, 0)),
        out_shape=jax.ShapeDtypeStruct((b, l, d), x.dtype),
        scratch_shapes=[pltpu.VMEM((16, 128), jnp.int32)],
    )(x, wrt, brm)


# speedup vs baseline: 4.1555x; 2.2529x over previous
import jax
import jax.numpy as jnp
from jax.experimental import pallas as pl
from jax.experimental.pallas import tpu as pltpu

_L = 2048
_D = 768
_K = 256
_S = 32
_NT = _L // _S


def _route_kernel(x_ref, wrt_ref, br_ref, out_ref, skey_scr):
    xb = x_ref[0]
    w = jnp.dot(xb, wrt_ref[...], preferred_element_type=jnp.float32)
    w = w + br_ref[0, 0]

    s = jax.lax.bitcast_convert_type(w, jnp.int32)
    skey = jnp.where(s < 0, s ^ jnp.int32(0x7FFFFFFF), s)
    skey_scr[...] = skey.reshape(16, 128)
    skey_r = skey_scr[...]

    kfv = jnp.full((1, 128), float(_K), jnp.float32)
    zero_i = jnp.zeros((1, 128), jnp.int32)
    int_min = jnp.full((1, 128), -(2 ** 31), jnp.int32)
    sub16 = jax.lax.broadcasted_iota(jnp.int32, (16, 128), 0)

    def cnt_single(pred):
        ps = jnp.sum(pred.astype(jnp.float32), axis=0, keepdims=True)
        for sh in (1, 2, 4, 8, 16, 32, 64):
            ps = ps + pltpu.roll(ps, sh, 1)
        return ps

    def round_digit(pred_fn, ok_fn, nc):
        acc = jnp.zeros((16, 128), jnp.float32)
        for c in range(1, nc):
            ps = jnp.sum(pred_fn(c).astype(jnp.float32), axis=0, keepdims=True)
            acc = jnp.where(sub16 == (c - 1), ps, acc)
        for sh in (1, 2, 4, 8, 16, 32, 64):
            acc = acc + pltpu.roll(acc, sh, 1)
        ok = ok_fn(acc) & (sub16 < (nc - 1))
        return jnp.sum(ok.astype(jnp.int32), axis=0, keepdims=True)

    thr = jnp.where(cnt_single(skey_r >= 0) >= kfv, zero_i, int_min)
    for shift, nc in [(28, 8), (24, 16), (20, 16), (16, 16), (12, 16),
                      (8, 16), (4, 16), (0, 16)]:
        d = round_digit(lambda c: skey_r >= thr + jnp.int32(c << shift),
                        lambda a: a >= kfv, nc)
        thr = thr + (d << shift)

    need = kfv - cnt_single(skey_r > thr)
    tie_r = skey_r == thr
    idx_r = (jax.lax.broadcasted_iota(jnp.int32, (16, 128), 0) * 128
             + jax.lax.broadcasted_iota(jnp.int32, (16, 128), 1))
    cut = zero_i
    for shift in (8, 4, 0):
        d = round_digit(
            lambda c: tie_r & (idx_r < cut + jnp.int32(c << shift)),
            lambda a: a <= need, 16)
        cut = cut + (d << shift)

    thr_s = thr[0:1, 0:1]
    cut_s = cut[0:1, 0:1]
    idx = jax.lax.broadcasted_iota(jnp.int32, (_L, 1), 0)
    mask = (skey > thr_s) | ((skey == thr_s) & (idx < cut_s))

    ir = jax.lax.broadcasted_iota(jnp.int32, (_S, _S), 0)
    ic = jax.lax.broadcasted_iota(jnp.int32, (_S, _S), 1)
    p = (ir + ic == _S - 1).astype(jnp.float32)
    zero = jnp.float32(0.0)
    for t in range(_NT):
        src = xb[_L - _S * (t + 1):_L - _S * t, :]
        flipped = jnp.dot(p, src, preferred_element_type=jnp.float32,
                          precision=jax.lax.Precision.HIGHEST)
        mb = mask[_S * t:_S * (t + 1), :]
        out_ref[0, _S * t:_S * (t + 1), :] = jnp.where(mb, flipped, zero)


def kernel(x, Wr, br, Wq, Wk, Wv):
    b, l, d = x.shape
    wrt = Wr.T
    brm = br.reshape(1, 1)
    return pl.pallas_call(
        _route_kernel,
        grid=(b,),
        in_specs=[
            pl.BlockSpec((1, l, d), lambda i: (i, 0, 0)),
            pl.BlockSpec((d, 1), lambda i: (0, 0)),
            pl.BlockSpec((1, 1), lambda i: (0, 0)),
        ],
        out_specs=pl.BlockSpec((1, l, d), lambda i: (i, 0, 0)),
        out_shape=jax.ShapeDtypeStruct((b, l, d), x.dtype),
        scratch_shapes=[pltpu.VMEM((16, 128), jnp.int32)],
    )(x, wrt, brm)
